# baseline (speedup 1.0000x reference)
import jax
import jax.numpy as jnp
from jax import lax
from jax.experimental import pallas as pl
from jax.experimental.pallas import tpu as pltpu
from jax.experimental.pallas import tpu_sc as plsc

_N = 10000
_E = 640000
_T = 12
_B = 2
_F = 90
_C = 256
_NW = 32
_EPW = _E // _NW
_L = 16
_NCHUNK = 25
_BLK = _N // _NCHUNK


def _sc_deg_body(dst_hbm, out_hbm, dst_v, deg_v):
    wid = lax.axis_index("s") * 2 + lax.axis_index("c")
    pltpu.sync_copy(dst_hbm.at[pl.ds(wid * _EPW, _EPW)], dst_v)

    zero16 = jnp.zeros((_L,), jnp.float32)

    def _zero(j, carry):
        deg_v[pl.ds(j * _L, _L)] = zero16
        return carry

    lax.fori_loop(0, _N // _L, _zero, 0)

    ones = jnp.ones((_L,), jnp.float32)

    def _step(i, carry):
        d = dst_v[pl.ds(i * _L, _L)]
        plsc.addupdate_scatter(deg_v, [d], ones)
        return carry

    lax.fori_loop(0, _EPW // _L, _step, 0)

    for nb in range(_NCHUNK):
        pltpu.sync_copy(deg_v.at[pl.ds(nb * _BLK, _BLK)], out_hbm.at[nb, wid])


_sc_deg_cache = []


def _get_sc_deg():
    if not _sc_deg_cache:
        _sc_deg_cache.append(pl.kernel(
            _sc_deg_body,
            out_type=jax.ShapeDtypeStruct((_NCHUNK, _NW, _BLK), jnp.float32),
            mesh=plsc.VectorSubcoreMesh(core_axis_name="c", subcore_axis_name="s",
                                        num_cores=2, num_subcores=16),
            compiler_params=pltpu.CompilerParams(needs_layout_passes=False,
                                                 use_tc_tiling_on_sc=False),
            scratch_types=[
                pltpu.VMEM((_EPW,), jnp.int32),
                pltpu.VMEM((_N,), jnp.float32),
            ],
        ))
    return _sc_deg_cache[0]


def _tc_dinv_body(fol_s, hp_r, dinv_r, df_r):
    f0 = fol_s[0]
    f1 = fol_s[1]
    df0 = jnp.float32(0.0)
    df1 = jnp.float32(0.0)
    for j in range(_NCHUNK):
        deg = jnp.sum(hp_r[j], axis=0, keepdims=True) + 1.0
        dv = 1.0 / jnp.sqrt(deg)
        nid = lax.broadcasted_iota(jnp.int32, (1, _BLK), 1) + j * _BLK
        df0 = df0 + jnp.sum(jnp.where(nid == f0, dv, 0.0))
        df1 = df1 + jnp.sum(jnp.where(nid == f1, dv, 0.0))
        dinv_r[pl.ds(j * _BLK, _BLK)] = dv.reshape((_BLK,))
    row = jnp.concatenate(
        [jnp.full((1, 1), df0, jnp.float32),
         jnp.full((1, 1), df1, jnp.float32),
         jnp.zeros((1, 126), jnp.float32)], axis=1)
    df_r[...] = jnp.zeros((8, 128), jnp.float32) + row


_tc_dinv = pl.pallas_call(
    _tc_dinv_body,
    in_specs=[pl.BlockSpec(memory_space=pltpu.SMEM),
              pl.BlockSpec(memory_space=pltpu.VMEM)],
    out_specs=[pl.BlockSpec(memory_space=pltpu.VMEM),
               pl.BlockSpec(memory_space=pltpu.VMEM)],
    out_shape=[jax.ShapeDtypeStruct((_N,), jnp.float32),
               jax.ShapeDtypeStruct((8, 128), jnp.float32)],
)


_CAP = _EPW + _L


def _sc_gather_body(src_hbm, dst_hbm, fol_hbm, dinv_hbm, x5_hbm, out_hbm,
                    src_v, dst_v, fol_v, dinv_v, l0_v, l1_v, tiles_v, acc_v, sem):
    wid = lax.axis_index("s") * 2 + lax.axis_index("c")
    pltpu.sync_copy(src_hbm.at[pl.ds(wid * _EPW, _EPW)], src_v)
    pltpu.sync_copy(dst_hbm.at[pl.ds(wid * _EPW, _EPW)], dst_v)
    pltpu.sync_copy(fol_hbm, fol_v)
    pltpu.sync_copy(dinv_hbm, dinv_v)

    zero16 = jnp.zeros((_L,), jnp.float32)
    for rrow in range(_B * _T):
        for c0 in range(8):
            acc_v[rrow, pl.ds(c0 * _L, _L)] = zero16

    f0 = fol_v[pl.ds(0, _L)]
    f1 = fol_v[pl.ds(_L, _L)]

    def _cstep(i, carry):
        nc0, nc1 = carry
        d = dst_v[pl.ds(i * _L, _L)]
        s = src_v[pl.ds(i * _L, _L)]
        m0 = d == f0
        m1 = d == f1
        plsc.store_compressed(l0_v.at[pl.ds(nc0, _L)], s, mask=m0)
        plsc.store_compressed(l1_v.at[pl.ds(nc1, _L)], s, mask=m1)
        nc0 = nc0 + plsc.all_reduce_population_count(m0)[0]
        nc1 = nc1 + plsc.all_reduce_population_count(m1)[0]
        return nc0, nc1

    nc0, nc1 = lax.fori_loop(0, _EPW // _L, _cstep,
                             (jnp.int32(0), jnp.int32(0)))

    lane = lax.broadcasted_iota(jnp.int32, (_L,), 0)
    is_w0 = jnp.zeros((_L,), jnp.int32) + wid == 0
    selmask = (lane == 0) & is_w0
    plsc.store_scatter(l0_v, [jnp.zeros((_L,), jnp.int32) + nc0], f0,
                       mask=selmask)
    plsc.store_scatter(l1_v, [jnp.zeros((_L,), jnp.int32) + nc1], f1,
                       mask=selmask)
    extra = jnp.where(wid == 0, jnp.int32(1), jnp.int32(0))
    nc0 = nc0 + extra
    nc1 = nc1 + extra

    lane6 = lane >= 6

    for bb in range(_B):
        lref = l0_v if bb == 0 else l1_v
        cnt = nc0 if bb == 0 else nc1
        nbatch = (cnt + (_L - 1)) // _L

        def _gbatch(ib, carry, lref=lref, cnt=cnt, bb=bb):
            ebase = ib * _L
            nvec = lref[pl.ds(ebase, _L)]
            valid = (lane + ebase) < cnt
            wv = plsc.load_gather(dinv_v, [nvec], mask=valid)
            for ln in range(_L):
                n = nvec[ln]
                w = wv[ln]
                g = n // 8
                r = n % 8

                @pl.when(ebase + ln < cnt)
                def _do(n=n, w=w, g=g, r=r, bb=bb):
                    cp = pltpu.make_async_copy(
                        x5_hbm.at[bb, :, g], tiles_v, sem)
                    cp.start()
                    cp.wait()

                    def _acc_t(t, carry2, w=w, r=r, bb=bb):
                        rrow = bb * _T + t
                        for c0 in (0, 16, 32, 48, 64):
                            acc_v[rrow, pl.ds(c0, _L)] = (
                                acc_v[rrow, pl.ds(c0, _L)]
                                + w * tiles_v[t, r, pl.ds(c0, _L)])
                        tail = w * tiles_v[t, r, pl.ds(_F - _L, _L)]
                        tail = jnp.where(lane6, tail, jnp.zeros((_L,), jnp.float32))
                        acc_v[rrow, pl.ds(_F - _L, _L)] = (
                            acc_v[rrow, pl.ds(_F - _L, _L)] + tail)
                        return carry2

                    lax.fori_loop(0, _T, _acc_t, 0)
            return carry

        lax.fori_loop(0, nbatch, _gbatch, 0)

    pltpu.sync_copy(acc_v, out_hbm.at[wid])


_sc_gather_cache = []


def _get_sc_gather():
    if not _sc_gather_cache:
        _sc_gather_cache.append(pl.kernel(
            _sc_gather_body,
            out_type=jax.ShapeDtypeStruct((_NW, _B * _T, 128), jnp.float32),
            mesh=plsc.VectorSubcoreMesh(core_axis_name="c", subcore_axis_name="s",
                                        num_cores=2, num_subcores=16),
            compiler_params=pltpu.CompilerParams(needs_layout_passes=False,
                                                 use_tc_tiling_on_sc=True),
            scratch_types=[
                pltpu.VMEM((_EPW,), jnp.int32),
                pltpu.VMEM((_EPW,), jnp.int32),
                pltpu.VMEM((2 * _L,), jnp.int32),
                pltpu.VMEM((_N,), jnp.float32),
                pltpu.VMEM((_CAP,), jnp.int32),
                pltpu.VMEM((_CAP,), jnp.int32),
                pltpu.VMEM((_T, 8, _F), jnp.float32),
                pltpu.VMEM((_B * _T, 128), jnp.float32),
                pltpu.SemaphoreType.DMA,
            ],
        ))
    return _sc_gather_cache[0]


def _tc2_body(p_r, df_r, att_r, wz_r, bz_r, wh_r, bh_r,
              lzw_r, lzb_r, lhw_r, lhb_r, r1w_r, r1b_r, r2w_r, r2b_r,
              r3w_r, r3b_r, out_r):
    accsum = p_r[0]
    for w in range(1, _NW):
        accsum = accsum + p_r[w]
    acc = accsum[:, 0:_F]
    df0 = jnp.sum(df_r[0:1, 0:1])
    df1 = jnp.sum(df_r[0:1, 1:2])
    sc0 = jnp.zeros((_T, 1), jnp.float32) + df0
    sc1 = jnp.zeros((_T, 1), jnp.float32) + df1
    sc_col = jnp.concatenate([sc0, sc1], axis=0)
    agg = acc * sc_col
    gz = jnp.dot(agg, wz_r[...], preferred_element_type=jnp.float32) + bz_r[...]
    z = jax.nn.sigmoid(jnp.dot(gz, lzw_r[...], preferred_element_type=jnp.float32) + lzb_r[...])
    gh = jnp.dot(agg, wh_r[...], preferred_element_type=jnp.float32) + bh_r[...]
    ht = jnp.tanh(jnp.dot(gh, lhw_r[...], preferred_element_type=jnp.float32) + lhb_r[...])
    u = (1.0 - z) * ht
    p = jax.nn.softmax(att_r[...], axis=-1)
    z12 = jnp.zeros((1, _T), jnp.float32)
    pmat = jnp.concatenate(
        [jnp.concatenate([p, z12], axis=1),
         jnp.concatenate([z12, p], axis=1)], axis=0)
    h = jnp.dot(pmat, u, preferred_element_type=jnp.float32)
    h = jnp.dot(h, r1w_r[...], preferred_element_type=jnp.float32) + r1b_r[...]
    h = jnp.where(h > 0, h, 0.01 * h)
    h = jnp.dot(h, r2w_r[...], preferred_element_type=jnp.float32) + r2b_r[...]
    h = jnp.where(h > 0, h, 0.01 * h)
    o = jnp.dot(h, r3w_r[...], preferred_element_type=jnp.float32) + r3b_r[...]
    out_r[...] = 4.0 * jax.nn.sigmoid(o) + 1.0


_tc2 = pl.pallas_call(
    _tc2_body,
    in_specs=[pl.BlockSpec(memory_space=pltpu.VMEM)] * 17,
    out_specs=pl.BlockSpec(memory_space=pltpu.VMEM),
    out_shape=jax.ShapeDtypeStruct((_B, 5), jnp.float32),
)


def kernel(x, edge_index, follower_ids, attention, W_z, b_z, W_r, b_r, W_h, b_h,
           lz_W, lz_b, lr_W, lr_b, lh_W, lh_b, r1_W, r1_b, r2_W, r2_b, r3_W, r3_b):
    ei = edge_index[0]
    src = ei[0]
    dst = ei[1]
    fol32 = jnp.repeat(follower_ids, _L)
    x5 = x.reshape(_B, _T, _N // 8, 8, _F)
    degp = _get_sc_deg()(dst)
    dinv, df = _tc_dinv(follower_ids, degp)
    partials = _get_sc_gather()(src, dst, fol32, dinv, x5)
    return _tc2(
        partials, df, attention.reshape(1, _T),
        W_z, b_z.reshape(1, _C), W_h, b_h.reshape(1, _C),
        lz_W[:_C], lz_b.reshape(1, _C), lh_W[:_C], lh_b.reshape(1, _C),
        r1_W, r1_b.reshape(1, 64), r2_W, r2_b.reshape(1, 32),
        r3_W, r3_b.reshape(1, 5))


# speedup vs baseline: 2.0688x; 2.0688x over previous
import jax
import jax.numpy as jnp
from jax import lax
from jax.experimental import pallas as pl
from jax.experimental.pallas import tpu as pltpu
from jax.experimental.pallas import tpu_sc as plsc

_N = 10000
_E = 640000
_T = 12
_B = 2
_F = 90
_C = 256
_NW = 32
_EPW = _E // _NW
_L = 16
_NCHUNK = 25
_BLK = _N // _NCHUNK


def _sc_deg_body(dst_hbm, out_hbm, dst_v, deg_v):
    wid = lax.axis_index("s") * 2 + lax.axis_index("c")
    pltpu.sync_copy(dst_hbm.at[pl.ds(wid * _EPW, _EPW)], dst_v)

    zero16 = jnp.zeros((_L,), jnp.float32)

    def _zero(j, carry):
        deg_v[pl.ds(j * _L, _L)] = zero16
        return carry

    lax.fori_loop(0, _N // _L, _zero, 0)

    ones = jnp.ones((_L,), jnp.float32)

    def _step(i, carry):
        d = dst_v[pl.ds(i * _L, _L)]
        plsc.addupdate_scatter(deg_v, [d], ones)
        return carry

    lax.fori_loop(0, _EPW // _L, _step, 0)

    for nb in range(_NCHUNK):
        pltpu.sync_copy(deg_v.at[pl.ds(nb * _BLK, _BLK)], out_hbm.at[nb, wid])


_sc_deg_cache = []


def _get_sc_deg():
    if not _sc_deg_cache:
        _sc_deg_cache.append(pl.kernel(
            _sc_deg_body,
            out_type=jax.ShapeDtypeStruct((_NCHUNK, _NW, _BLK), jnp.float32),
            mesh=plsc.VectorSubcoreMesh(core_axis_name="c", subcore_axis_name="s",
                                        num_cores=2, num_subcores=16),
            compiler_params=pltpu.CompilerParams(needs_layout_passes=False,
                                                 use_tc_tiling_on_sc=False),
            scratch_types=[
                pltpu.VMEM((_EPW,), jnp.int32),
                pltpu.VMEM((_N,), jnp.float32),
            ],
        ))
    return _sc_deg_cache[0]


def _tc_dinv_body(fol_s, hp_r, dinv_r, df_r):
    f0 = fol_s[0]
    f1 = fol_s[1]
    df0 = jnp.float32(0.0)
    df1 = jnp.float32(0.0)
    for j in range(_NCHUNK):
        deg = jnp.sum(hp_r[j], axis=0, keepdims=True) + 1.0
        dv = 1.0 / jnp.sqrt(deg)
        nid = lax.broadcasted_iota(jnp.int32, (1, _BLK), 1) + j * _BLK
        df0 = df0 + jnp.sum(jnp.where(nid == f0, dv, 0.0))
        df1 = df1 + jnp.sum(jnp.where(nid == f1, dv, 0.0))
        dinv_r[pl.ds(j * _BLK, _BLK)] = dv.reshape((_BLK,))
    row = jnp.concatenate(
        [jnp.full((1, 1), df0, jnp.float32),
         jnp.full((1, 1), df1, jnp.float32),
         jnp.zeros((1, 126), jnp.float32)], axis=1)
    df_r[...] = jnp.zeros((8, 128), jnp.float32) + row


_tc_dinv = pl.pallas_call(
    _tc_dinv_body,
    in_specs=[pl.BlockSpec(memory_space=pltpu.SMEM),
              pl.BlockSpec(memory_space=pltpu.VMEM)],
    out_specs=[pl.BlockSpec(memory_space=pltpu.VMEM),
               pl.BlockSpec(memory_space=pltpu.VMEM)],
    out_shape=[jax.ShapeDtypeStruct((_N,), jnp.float32),
               jax.ShapeDtypeStruct((8, 128), jnp.float32)],
)


_CAP = _EPW + _L


def _sc_gather_body(src_hbm, dst_hbm, fol_hbm, dinv_hbm, x5_hbm, out_hbm,
                    src_v, dst_v, fol_v, dinv_v, l0_v, l1_v, tiles_v, acc_v, sem):
    wid = lax.axis_index("s") * 2 + lax.axis_index("c")
    pltpu.sync_copy(src_hbm.at[pl.ds(wid * _EPW, _EPW)], src_v)
    pltpu.sync_copy(dst_hbm.at[pl.ds(wid * _EPW, _EPW)], dst_v)
    pltpu.sync_copy(fol_hbm, fol_v)
    pltpu.sync_copy(dinv_hbm, dinv_v)

    zero16 = jnp.zeros((_L,), jnp.float32)
    for rrow in range(_B * _T):
        for c0 in range(8):
            acc_v[rrow, pl.ds(c0 * _L, _L)] = zero16

    f0 = fol_v[pl.ds(0, _L)]
    f1 = fol_v[pl.ds(_L, _L)]

    def _cstep(i, carry):
        nc0, nc1 = carry
        d = dst_v[pl.ds(i * _L, _L)]
        s = src_v[pl.ds(i * _L, _L)]
        m0 = d == f0
        m1 = d == f1
        plsc.store_compressed(l0_v.at[pl.ds(nc0, _L)], s, mask=m0)
        plsc.store_compressed(l1_v.at[pl.ds(nc1, _L)], s, mask=m1)
        nc0 = nc0 + plsc.all_reduce_population_count(m0)[0]
        nc1 = nc1 + plsc.all_reduce_population_count(m1)[0]
        return nc0, nc1

    nc0, nc1 = lax.fori_loop(0, _EPW // _L, _cstep,
                             (jnp.int32(0), jnp.int32(0)))

    lane = lax.broadcasted_iota(jnp.int32, (_L,), 0)
    is_w0 = jnp.zeros((_L,), jnp.int32) + wid == 0
    selmask = (lane == 0) & is_w0
    plsc.store_scatter(l0_v, [jnp.zeros((_L,), jnp.int32) + nc0], f0,
                       mask=selmask)
    plsc.store_scatter(l1_v, [jnp.zeros((_L,), jnp.int32) + nc1], f1,
                       mask=selmask)
    extra = jnp.where(wid == 0, jnp.int32(1), jnp.int32(0))
    nc0 = nc0 + extra
    nc1 = nc1 + extra

    lane6 = lane >= 6

    for bb in range(_B):
        lref = l0_v if bb == 0 else l1_v
        cnt = nc0 if bb == 0 else nc1
        nbatch = (cnt + (_L - 1)) // _L

        def _gbatch(ib, carry, lref=lref, cnt=cnt, bb=bb):
            ebase = ib * _L
            nvec = lref[pl.ds(ebase, _L)]
            valid = (lane + ebase) < cnt
            wv = plsc.load_gather(dinv_v, [nvec], mask=valid)
            for ln in range(_L):
                n = nvec[ln]
                w = wv[ln]
                g = n // 8
                r = n % 8

                @pl.when(ebase + ln < cnt)
                def _do(n=n, w=w, g=g, r=r, bb=bb):
                    cp = pltpu.make_async_copy(
                        x5_hbm.at[bb, :, pl.ds(g * 8, 8), :], tiles_v, sem)
                    cp.start()
                    cp.wait()

                    def _acc_t(t, carry2, w=w, r=r, bb=bb):
                        rrow = bb * _T + t
                        for c0 in (0, 16, 32, 48, 64):
                            acc_v[rrow, pl.ds(c0, _L)] = (
                                acc_v[rrow, pl.ds(c0, _L)]
                                + w * tiles_v[t, r, pl.ds(c0, _L)])
                        tail = w * tiles_v[t, r, pl.ds(_F - _L, _L)]
                        tail = jnp.where(lane6, tail, jnp.zeros((_L,), jnp.float32))
                        acc_v[rrow, pl.ds(_F - _L, _L)] = (
                            acc_v[rrow, pl.ds(_F - _L, _L)] + tail)
                        return carry2

                    lax.fori_loop(0, _T, _acc_t, 0)
            return carry

        lax.fori_loop(0, nbatch, _gbatch, 0)

    pltpu.sync_copy(acc_v, out_hbm.at[wid])


_sc_gather_cache = []


def _get_sc_gather():
    if not _sc_gather_cache:
        _sc_gather_cache.append(pl.kernel(
            _sc_gather_body,
            out_type=jax.ShapeDtypeStruct((_NW, _B * _T, 128), jnp.float32),
            mesh=plsc.VectorSubcoreMesh(core_axis_name="c", subcore_axis_name="s",
                                        num_cores=2, num_subcores=16),
            compiler_params=pltpu.CompilerParams(needs_layout_passes=False,
                                                 use_tc_tiling_on_sc=True),
            scratch_types=[
                pltpu.VMEM((_EPW,), jnp.int32),
                pltpu.VMEM((_EPW,), jnp.int32),
                pltpu.VMEM((2 * _L,), jnp.int32),
                pltpu.VMEM((_N,), jnp.float32),
                pltpu.VMEM((_CAP,), jnp.int32),
                pltpu.VMEM((_CAP,), jnp.int32),
                pltpu.VMEM((_T, 8, _F), jnp.float32),
                pltpu.VMEM((_B * _T, 128), jnp.float32),
                pltpu.SemaphoreType.DMA,
            ],
        ))
    return _sc_gather_cache[0]


def _tc2_body(p_r, df_r, att_r, wz_r, bz_r, wh_r, bh_r,
              lzw_r, lzb_r, lhw_r, lhb_r, r1w_r, r1b_r, r2w_r, r2b_r,
              r3w_r, r3b_r, out_r):
    accsum = p_r[0]
    for w in range(1, _NW):
        accsum = accsum + p_r[w]
    acc = accsum[:, 0:_F]
    df0 = jnp.sum(df_r[0:1, 0:1])
    df1 = jnp.sum(df_r[0:1, 1:2])
    sc0 = jnp.zeros((_T, 1), jnp.float32) + df0
    sc1 = jnp.zeros((_T, 1), jnp.float32) + df1
    sc_col = jnp.concatenate([sc0, sc1], axis=0)
    agg = acc * sc_col
    gz = jnp.dot(agg, wz_r[...], preferred_element_type=jnp.float32) + bz_r[...]
    z = jax.nn.sigmoid(jnp.dot(gz, lzw_r[...], preferred_element_type=jnp.float32) + lzb_r[...])
    gh = jnp.dot(agg, wh_r[...], preferred_element_type=jnp.float32) + bh_r[...]
    ht = jnp.tanh(jnp.dot(gh, lhw_r[...], preferred_element_type=jnp.float32) + lhb_r[...])
    u = (1.0 - z) * ht
    p = jax.nn.softmax(att_r[...], axis=-1)
    z12 = jnp.zeros((1, _T), jnp.float32)
    pmat = jnp.concatenate(
        [jnp.concatenate([p, z12], axis=1),
         jnp.concatenate([z12, p], axis=1)], axis=0)
    h = jnp.dot(pmat, u, preferred_element_type=jnp.float32)
    h = jnp.dot(h, r1w_r[...], preferred_element_type=jnp.float32) + r1b_r[...]
    h = jnp.where(h > 0, h, 0.01 * h)
    h = jnp.dot(h, r2w_r[...], preferred_element_type=jnp.float32) + r2b_r[...]
    h = jnp.where(h > 0, h, 0.01 * h)
    o = jnp.dot(h, r3w_r[...], preferred_element_type=jnp.float32) + r3b_r[...]
    out_r[...] = 4.0 * jax.nn.sigmoid(o) + 1.0


_tc2 = pl.pallas_call(
    _tc2_body,
    in_specs=[pl.BlockSpec(memory_space=pltpu.VMEM)] * 17,
    out_specs=pl.BlockSpec(memory_space=pltpu.VMEM),
    out_shape=jax.ShapeDtypeStruct((_B, 5), jnp.float32),
)


def kernel(x, edge_index, follower_ids, attention, W_z, b_z, W_r, b_r, W_h, b_h,
           lz_W, lz_b, lr_W, lr_b, lh_W, lh_b, r1_W, r1_b, r2_W, r2_b, r3_W, r3_b):
    ei = edge_index[0]
    src = ei[0]
    dst = ei[1]
    fol32 = jnp.repeat(follower_ids, _L)
    degp = _get_sc_deg()(dst)
    dinv, df = _tc_dinv(follower_ids, degp)
    partials = _get_sc_gather()(src, dst, fol32, dinv, x)
    return _tc2(
        partials, df, attention.reshape(1, _T),
        W_z, b_z.reshape(1, _C), W_h, b_h.reshape(1, _C),
        lz_W[:_C], lz_b.reshape(1, _C), lh_W[:_C], lh_b.reshape(1, _C),
        r1_W, r1_b.reshape(1, 64), r2_W, r2_b.reshape(1, 32),
        r3_W, r3_b.reshape(1, 5))


# speedup vs baseline: 2.0892x; 1.0099x over previous
import jax
import jax.numpy as jnp
from jax import lax
from jax.experimental import pallas as pl
from jax.experimental.pallas import tpu as pltpu
from jax.experimental.pallas import tpu_sc as plsc

_N = 10000
_E = 640000
_T = 12
_B = 2
_F = 90
_C = 256
_NW = 32
_EPW = _E // _NW
_L = 16
_DR = 80
_CAP = _EPW + _L


def _qrsqrt(v):
    i = plsc.bitcast(v, jnp.int32)
    i = jnp.int32(0x5F3759DF) - lax.shift_right_logical(i, 1)
    y = plsc.bitcast(i, jnp.float32)
    for _ in range(3):
        y = y * (1.5 - 0.5 * v * y * y)
    return y


def _sc_deg_body(dst_hbm, out_hbm, dst_v, deg3_v, idx_v, shared):
    cid = lax.axis_index("c")
    sid = lax.axis_index("s")
    wid = sid * 2 + cid
    pltpu.sync_copy(dst_hbm.at[pl.ds(wid * _EPW, _EPW)], dst_v)

    zero16 = jnp.zeros((_L,), jnp.float32)

    def _zero(j, carry):
        for c0 in range(8):
            deg3_v[j, pl.ds(c0 * _L, _L)] = zero16
        return carry

    lax.fori_loop(0, _DR, _zero, 0)
    for j in range(_DR // _L):
        idx_v[pl.ds(j * _L, _L)] = (
            lax.broadcasted_iota(jnp.int32, (_L,), 0) + j * _L)

    @pl.when(sid == 0)
    def _init_shared():
        pltpu.sync_copy(deg3_v, shared)

    plsc.subcore_barrier()

    ones = jnp.ones((_L,), jnp.float32)

    def _step(i, carry):
        d = dst_v[pl.ds(i * _L, _L)]
        plsc.addupdate_scatter(
            deg3_v,
            [lax.shift_right_logical(d, 7), d & jnp.int32(127)], ones)
        return carry

    lax.fori_loop(0, _EPW // _L, _step, 0)

    pltpu.sync_copy(deg3_v, shared.at[idx_v], add=True)
    plsc.subcore_barrier()

    @pl.when(sid == 0)
    def _writeout():
        pltpu.sync_copy(shared, out_hbm.at[cid])


_sc_deg_cache = []


def _get_sc_deg():
    if not _sc_deg_cache:
        _sc_deg_cache.append(pl.kernel(
            _sc_deg_body,
            out_type=jax.ShapeDtypeStruct((2, _DR, 128), jnp.float32),
            mesh=plsc.VectorSubcoreMesh(core_axis_name="c", subcore_axis_name="s",
                                        num_cores=2, num_subcores=16),
            compiler_params=pltpu.CompilerParams(needs_layout_passes=False,
                                                 use_tc_tiling_on_sc=False),
            scratch_types=[
                pltpu.VMEM((_EPW,), jnp.int32),
                pltpu.VMEM((_DR, 128), jnp.float32),
                pltpu.VMEM((_DR,), jnp.int32),
                pltpu.VMEM_SHARED((_DR, 128), jnp.float32),
            ],
        ))
    return _sc_deg_cache[0]


def _sc_gather_body(src_hbm, dst_hbm, fol_hbm, degp_hbm, x_hbm, out_hbm,
                    src_v, dst_v, fol_v, degA_v, degB_v, l0_v, l1_v,
                    tiles_v, acc_v, sem):
    wid = lax.axis_index("s") * 2 + lax.axis_index("c")
    pltpu.sync_copy(src_hbm.at[pl.ds(wid * _EPW, _EPW)], src_v)
    pltpu.sync_copy(dst_hbm.at[pl.ds(wid * _EPW, _EPW)], dst_v)
    pltpu.sync_copy(fol_hbm, fol_v)
    pltpu.sync_copy(degp_hbm.at[0], degA_v)
    pltpu.sync_copy(degp_hbm.at[1], degB_v)

    zero16 = jnp.zeros((_L,), jnp.float32)
    for rrow in range(_B * _T):
        for c0 in range(8):
            acc_v[rrow, pl.ds(c0 * _L, _L)] = zero16

    f0 = fol_v[pl.ds(0, _L)]
    f1 = fol_v[pl.ds(_L, _L)]

    def _cstep(i, carry):
        nc0, nc1 = carry
        d = dst_v[pl.ds(i * _L, _L)]
        s = src_v[pl.ds(i * _L, _L)]
        m0 = d == f0
        m1 = d == f1
        plsc.store_compressed(l0_v.at[pl.ds(nc0, _L)], s, mask=m0)
        plsc.store_compressed(l1_v.at[pl.ds(nc1, _L)], s, mask=m1)
        nc0 = nc0 + plsc.all_reduce_population_count(m0)[0]
        nc1 = nc1 + plsc.all_reduce_population_count(m1)[0]
        return nc0, nc1

    nc0, nc1 = lax.fori_loop(0, _EPW // _L, _cstep,
                             (jnp.int32(0), jnp.int32(0)))

    lane = lax.broadcasted_iota(jnp.int32, (_L,), 0)
    is_w0 = jnp.zeros((_L,), jnp.int32) + wid == 0
    selmask = (lane == 0) & is_w0
    plsc.store_scatter(l0_v, [jnp.zeros((_L,), jnp.int32) + nc0], f0,
                       mask=selmask)
    plsc.store_scatter(l1_v, [jnp.zeros((_L,), jnp.int32) + nc1], f1,
                       mask=selmask)
    extra = jnp.where(wid == 0, jnp.int32(1), jnp.int32(0))
    nc0 = nc0 + extra
    nc1 = nc1 + extra

    lane6 = lane >= 6

    def _deg_of(nvec, mask):
        rows = lax.shift_right_logical(nvec, 7)
        cols = nvec & jnp.int32(127)
        da = plsc.load_gather(degA_v, [rows, cols], mask=mask)
        db = plsc.load_gather(degB_v, [rows, cols], mask=mask)
        return da + db + 1.0

    for bb in range(_B):
        lref = l0_v if bb == 0 else l1_v
        cnt = nc0 if bb == 0 else nc1
        fvec = f0 if bb == 0 else f1
        nbatch = (cnt + (_L - 1)) // _L

        def _gbatch(ib, carry, lref=lref, cnt=cnt, bb=bb):
            ebase = ib * _L
            nvec = lref[pl.ds(ebase, _L)]
            valid = (lane + ebase) < cnt
            wv = _qrsqrt(_deg_of(nvec, valid))
            for ln in range(_L):
                n = nvec[ln]
                w = wv[ln]
                g = n // 8
                r = n % 8

                @pl.when(ebase + ln < cnt)
                def _do(n=n, w=w, g=g, r=r, bb=bb):
                    cp = pltpu.make_async_copy(
                        x_hbm.at[bb, :, pl.ds(g * 8, 8), :], tiles_v, sem)
                    cp.start()
                    cp.wait()

                    def _acc_t(t, carry2, w=w, r=r, bb=bb):
                        rrow = bb * _T + t
                        for c0 in (0, 16, 32, 48, 64):
                            acc_v[rrow, pl.ds(c0, _L)] = (
                                acc_v[rrow, pl.ds(c0, _L)]
                                + w * tiles_v[t, r, pl.ds(c0, _L)])
                        tail = w * tiles_v[t, r, pl.ds(_F - _L, _L)]
                        tail = jnp.where(lane6, tail, jnp.zeros((_L,), jnp.float32))
                        acc_v[rrow, pl.ds(_F - _L, _L)] = (
                            acc_v[rrow, pl.ds(_F - _L, _L)] + tail)
                        return carry2

                    lax.fori_loop(0, _T, _acc_t, 0)
            return carry

        lax.fori_loop(0, nbatch, _gbatch, 0)

        dfv = _qrsqrt(_deg_of(fvec, None))
        for t in range(_T):
            for c0 in range(8):
                acc_v[bb * _T + t, pl.ds(c0 * _L, _L)] = (
                    acc_v[bb * _T + t, pl.ds(c0 * _L, _L)] * dfv)

    pltpu.sync_copy(acc_v, out_hbm.at[wid])


_sc_gather_cache = []


def _get_sc_gather():
    if not _sc_gather_cache:
        _sc_gather_cache.append(pl.kernel(
            _sc_gather_body,
            out_type=jax.ShapeDtypeStruct((_NW, _B * _T, 128), jnp.float32),
            mesh=plsc.VectorSubcoreMesh(core_axis_name="c", subcore_axis_name="s",
                                        num_cores=2, num_subcores=16),
            compiler_params=pltpu.CompilerParams(needs_layout_passes=False,
                                                 use_tc_tiling_on_sc=True),
            scratch_types=[
                pltpu.VMEM((_EPW,), jnp.int32),
                pltpu.VMEM((_EPW,), jnp.int32),
                pltpu.VMEM((2 * _L,), jnp.int32),
                pltpu.VMEM((_DR, 128), jnp.float32),
                pltpu.VMEM((_DR, 128), jnp.float32),
                pltpu.VMEM((_CAP,), jnp.int32),
                pltpu.VMEM((_CAP,), jnp.int32),
                pltpu.VMEM((_T, 8, _F), jnp.float32),
                pltpu.VMEM((_B * _T, 128), jnp.float32),
                pltpu.SemaphoreType.DMA,
            ],
        ))
    return _sc_gather_cache[0]


def _tc2_body(p_r, att_r, wz_r, bz_r, wh_r, bh_r,
              lzw_r, lzb_r, lhw_r, lhb_r, r1w_r, r1b_r, r2w_r, r2b_r,
              r3w_r, r3b_r, out_r):
    accsum = p_r[0]
    for w in range(1, _NW):
        accsum = accsum + p_r[w]
    agg = accsum[:, 0:_F]
    gz = jnp.dot(agg, wz_r[...], preferred_element_type=jnp.float32) + bz_r[...]
    z = jax.nn.sigmoid(jnp.dot(gz, lzw_r[...], preferred_element_type=jnp.float32) + lzb_r[...])
    gh = jnp.dot(agg, wh_r[...], preferred_element_type=jnp.float32) + bh_r[...]
    ht = jnp.tanh(jnp.dot(gh, lhw_r[...], preferred_element_type=jnp.float32) + lhb_r[...])
    u = (1.0 - z) * ht
    p = jax.nn.softmax(att_r[...], axis=-1)
    z12 = jnp.zeros((1, _T), jnp.float32)
    pmat = jnp.concatenate(
        [jnp.concatenate([p, z12], axis=1),
         jnp.concatenate([z12, p], axis=1)], axis=0)
    h = jnp.dot(pmat, u, preferred_element_type=jnp.float32)
    h = jnp.dot(h, r1w_r[...], preferred_element_type=jnp.float32) + r1b_r[...]
    h = jnp.where(h > 0, h, 0.01 * h)
    h = jnp.dot(h, r2w_r[...], preferred_element_type=jnp.float32) + r2b_r[...]
    h = jnp.where(h > 0, h, 0.01 * h)
    o = jnp.dot(h, r3w_r[...], preferred_element_type=jnp.float32) + r3b_r[...]
    out_r[...] = 4.0 * jax.nn.sigmoid(o) + 1.0


_tc2 = pl.pallas_call(
    _tc2_body,
    in_specs=[pl.BlockSpec(memory_space=pltpu.VMEM)] * 16,
    out_specs=pl.BlockSpec(memory_space=pltpu.VMEM),
    out_shape=jax.ShapeDtypeStruct((_B, 5), jnp.float32),
)


def kernel(x, edge_index, follower_ids, attention, W_z, b_z, W_r, b_r, W_h, b_h,
           lz_W, lz_b, lr_W, lr_b, lh_W, lh_b, r1_W, r1_b, r2_W, r2_b, r3_W, r3_b):
    ei = edge_index[0]
    src = ei[0]
    dst = ei[1]
    fol32 = jnp.repeat(follower_ids, _L)
    degp = _get_sc_deg()(dst)
    partials = _get_sc_gather()(src, dst, fol32, degp, x)
    return _tc2(
        partials, attention.reshape(1, _T),
        W_z, b_z.reshape(1, _C), W_h, b_h.reshape(1, _C),
        lz_W[:_C], lz_b.reshape(1, _C), lh_W[:_C], lh_b.reshape(1, _C),
        r1_W, r1_b.reshape(1, 64), r2_W, r2_b.reshape(1, 32),
        r3_W, r3_b.reshape(1, 5))
